# baseline (speedup 1.0000x reference)
import functools

import jax
import jax.numpy as jnp
from jax import lax
from jax.experimental import pallas as pl
from jax.experimental.pallas import tpu as pltpu
from jax.experimental.pallas import tpu_sc as plsc

N_NODES = 10000
D = 256
E = 160000

NC = 2
NS = 16
NW = NC * NS
L = 16
DPW = D // NW
NPAIR = DPW // 2
WPW = N_NODES * NPAIR

CH = 8000
NCHUNK = E // CH

_mesh = plsc.VectorSubcoreMesh(core_axis_name="c", subcore_axis_name="s")


@functools.partial(
    pl.kernel,
    mesh=_mesh,
    compiler_params=pltpu.CompilerParams(
        needs_layout_passes=False, use_tc_tiling_on_sc=False
    ),
    out_type=jax.ShapeDtypeStruct((NW * E,), jnp.float32),
    scratch_types=[
        pltpu.VMEM((WPW,), jnp.int32),
        pltpu.VMEM((CH,), jnp.int32),
        pltpu.VMEM((CH,), jnp.int32),
        pltpu.VMEM((CH,), jnp.int32),
        pltpu.VMEM((CH,), jnp.int32),
        pltpu.VMEM((CH,), jnp.float32),
        pltpu.VMEM((CH,), jnp.float32),
        pltpu.SemaphoreType.DMA,
        pltpu.SemaphoreType.DMA,
        pltpu.SemaphoreType.DMA,
        pltpu.SemaphoreType.DMA,
    ],
)
def _sc_partial_dots(
    zt_hbm, u_hbm, v_hbm, out_hbm,
    zt, ub0, ub1, vb0, vb1, pb0, pb1,
    in_sem0, in_sem1, out_sem0, out_sem1,
):
    cid = lax.axis_index("c")
    sid = lax.axis_index("s")
    wid = sid * NC + cid

    ubs = (ub0, ub1)
    vbs = (vb0, vb1)
    pbs = (pb0, pb1)
    in_sems = (in_sem0, in_sem1)
    out_sems = (out_sem0, out_sem1)

    pltpu.sync_copy(zt_hbm.at[pl.ds(wid * WPW, WPW)], zt)

    def start_in(c, b):
        pltpu.async_copy(u_hbm.at[pl.ds(c * CH, CH)], ubs[b], in_sems[b])
        pltpu.async_copy(v_hbm.at[pl.ds(c * CH, CH)], vbs[b], in_sems[b])

    def wait_in(b):
        pltpu.make_async_copy(u_hbm.at[pl.ds(0, CH)], ubs[b], in_sems[b]).wait()
        pltpu.make_async_copy(v_hbm.at[pl.ds(0, CH)], vbs[b], in_sems[b]).wait()

    def start_out(c, b):
        pltpu.async_copy(
            pbs[b], out_hbm.at[pl.ds(wid * E + c * CH, CH)], out_sems[b]
        )

    def wait_out(b):
        pltpu.make_async_copy(
            pbs[b], out_hbm.at[pl.ds(0, CH)], out_sems[b]
        ).wait()

    pcols = [jnp.full((L,), p * N_NODES, jnp.int32) for p in range(NPAIR)]
    zero = jnp.zeros((L,), jnp.float32)

    def compute_chunk(b):
        ub = ubs[b]
        vb = vbs[b]
        pb = pbs[b]

        def j_body(j, inner):
            off = j * L
            u16 = ub[pl.ds(off, L)]
            v16 = vb[pl.ds(off, L)]
            acc = zero
            for p in range(NPAIR):
                uw = plsc.load_gather(zt, [u16 + pcols[p]])
                vw = plsc.load_gather(zt, [v16 + pcols[p]])
                prod = plsc.bitcast(uw, jnp.bfloat16) * plsc.bitcast(
                    vw, jnp.bfloat16
                )
                lo, hi = plsc.unpack(prod, format=plsc.PackFormat.INTERLEAVED)
                acc = acc + lo
                acc = acc + hi
            pb[pl.ds(off, L)] = acc
            return inner

        lax.fori_loop(0, CH // L, j_body, 0)

    start_in(0, 0)

    def chunk_body(g):
        for b in range(2):
            c = g + b

            @pl.when(c + 1 < NCHUNK)
            def _():
                start_in(c + 1, 1 - b)

            wait_in(b)

            @pl.when(c >= 2)
            def _():
                wait_out(b)

            compute_chunk(b)
            start_out(c, b)

    pl.loop(0, NCHUNK, step=2)(chunk_body)

    wait_out(0)
    wait_out(1)


def _tc_sum_kernel(parts_ref, o_ref):
    o_ref[...] = jnp.sum(parts_ref[...], axis=0, keepdims=True)


_BLK = 16000


def _tc_sum(parts):
    out = pl.pallas_call(
        _tc_sum_kernel,
        grid=(E // _BLK,),
        in_specs=[pl.BlockSpec((NW, _BLK), lambda i: (0, i))],
        out_specs=pl.BlockSpec((1, _BLK), lambda i: (0, i)),
        out_shape=jax.ShapeDtypeStruct((1, E), jnp.float32),
    )(parts)
    return out[0]


def kernel(z, edges):
    edges = edges.astype(jnp.int32)
    u = edges[:, 0]
    v = edges[:, 1]
    zw = jax.lax.bitcast_convert_type(
        z.astype(jnp.bfloat16).reshape(N_NODES, D // 2, 2), jnp.int32
    )
    zt = zw.T.reshape(-1)
    parts = _sc_partial_dots(zt, u, v)
    return _tc_sum(parts.reshape(NW, E))
